# Initial kernel scaffold; baseline (speedup 1.0000x reference)
import jax
import jax.numpy as jnp
from jax.experimental import pallas as pl


def kernel(src, time, weekday, duration, emb_loc, minute_embed, hour_embed, weekday_embed, emb_duration):
    raise NotImplementedError("write your pallas kernel here")


# speedup vs baseline: 1.7051x; 1.7051x over previous
import functools

import jax
import jax.numpy as jnp
from jax import lax
from jax.experimental import pallas as pl
from jax.experimental.pallas import tpu as pltpu
from jax.experimental.pallas import tpu_sc as plsc

SEQ, B, D = 200, 1024, 64
N = SEQ * B
NC, NS = 2, 16
NW = NC * NS
ROWS_W = N // NW
CHUNK = 128
NCHUNK = ROWS_W // CHUNK
GROUPS = CHUNK // 16
CT_ROWS = 24 * 4 * 7


def _prep_body(time_ref, wd_ref, hour_ref, minute_ref, wde_ref,
               ct_idx_ref, ct_tab_ref):
    ct_idx_ref[...] = time_ref[...] * 7 + wd_ref[...]
    h = hour_ref[...]
    mi = minute_ref[...]
    w = wde_ref[...]
    ct_tab_ref[...] = (h[:, None, None, :] + mi[None, :, None, :]
                       + w[None, None, :, :])


_prep = pl.pallas_call(
    _prep_body,
    out_shape=(
        jax.ShapeDtypeStruct((SEQ, B), jnp.int32),
        jax.ShapeDtypeStruct((24, 4, 7, D), jnp.float32),
    ),
)


def _sc_body(loc_hbm, ct_tab_hbm, dur_tab_hbm, src_hbm, ct_hbm, dur_hbm,
             out_hbm, ct_v, durt_v, src_v, ctv, durv, rows_v, sem):
    wid = lax.axis_index("s") * NC + lax.axis_index("c")
    base_w = wid * ROWS_W
    pltpu.sync_copy(ct_tab_hbm, ct_v)
    pltpu.sync_copy(dur_tab_hbm, durt_v)

    def chunk_body(c, carry):
        base = base_w + c * CHUNK
        pltpu.sync_copy(src_hbm.at[pl.ds(base, CHUNK)], src_v)
        pltpu.sync_copy(ct_hbm.at[pl.ds(base, CHUNK)], ctv)
        pltpu.sync_copy(dur_hbm.at[pl.ds(base, CHUNK)], durv)
        pltpu.async_copy(loc_hbm.at[src_v], rows_v, sem).wait()

        def group_body(g, gcarry):
            gb = g * 16
            src16 = src_v[pl.ds(gb, 16)]
            ct16 = ctv[pl.ds(gb, 16)]
            dur16 = durv[pl.ds(gb, 16)]
            pad = src16 == 0
            rows16 = gb + lax.iota(jnp.int32, 16)
            for d in range(D):
                dsp = jnp.full((16,), d, jnp.int32)
                s = (plsc.load_gather(ct_v, [ct16, dsp])
                     + plsc.load_gather(durt_v, [dur16, dsp]))
                loc = plsc.load_gather(rows_v, [rows16, dsp])
                val = s + jnp.where(pad, 0.0, loc)
                plsc.store_scatter(rows_v, [rows16, dsp], val)
            return gcarry

        lax.fori_loop(0, GROUPS, group_body, 0)
        pltpu.sync_copy(rows_v, out_hbm.at[pl.ds(base, CHUNK)])
        return carry

    lax.fori_loop(0, NCHUNK, chunk_body, 0)


_sc_embed = functools.partial(
    pl.kernel,
    out_type=jax.ShapeDtypeStruct((N, D), jnp.float32),
    mesh=plsc.VectorSubcoreMesh(core_axis_name="c", subcore_axis_name="s"),
    compiler_params=pltpu.CompilerParams(needs_layout_passes=False,
                                         use_tc_tiling_on_sc=False),
    scratch_types=[
        pltpu.VMEM((CT_ROWS, D), jnp.float32),
        pltpu.VMEM((96, D), jnp.float32),
        pltpu.VMEM((CHUNK,), jnp.int32),
        pltpu.VMEM((CHUNK,), jnp.int32),
        pltpu.VMEM((CHUNK,), jnp.int32),
        pltpu.VMEM((CHUNK, D), jnp.float32),
        pltpu.SemaphoreType.DMA,
    ],
)(_sc_body)


def kernel(src, time, weekday, duration, emb_loc, minute_embed, hour_embed,
           weekday_embed, emb_duration):
    ct_idx, ct_tab4 = _prep(time.astype(jnp.int32), weekday.astype(jnp.int32),
                            hour_embed, minute_embed, weekday_embed)
    ct_tab = ct_tab4.reshape(CT_ROWS, D)
    out = _sc_embed(emb_loc, ct_tab, emb_duration,
                    src.reshape(N).astype(jnp.int32),
                    ct_idx.reshape(N),
                    duration.reshape(N).astype(jnp.int32))
    return out.reshape(SEQ, B, D)


# speedup vs baseline: 1.8304x; 1.0735x over previous
import functools

import jax
import jax.numpy as jnp
from jax import lax
from jax.experimental import pallas as pl
from jax.experimental.pallas import tpu as pltpu
from jax.experimental.pallas import tpu_sc as plsc

SEQ, B, D = 200, 1024, 64
N = SEQ * B
NC, NS = 2, 16
NW = NC * NS
ROWS_W = N // NW
CHUNK = 128
NCHUNK = ROWS_W // CHUNK
GROUPS = CHUNK // 16
CT_ROWS = 24 * 4 * 7


def _prep_body(time_ref, wd_ref, hour_ref, minute_ref, wde_ref,
               ct_idx_ref, ct_tab_ref):
    ct_idx_ref[...] = time_ref[...] * 7 + wd_ref[...]
    h = hour_ref[...]
    mi = minute_ref[...]
    w = wde_ref[...]
    ct_tab_ref[...] = (h[:, None, None, :] + mi[None, :, None, :]
                       + w[None, None, :, :])


_prep = pl.pallas_call(
    _prep_body,
    out_shape=(
        jax.ShapeDtypeStruct((SEQ, B), jnp.int32),
        jax.ShapeDtypeStruct((24, 4, 7, D), jnp.float32),
    ),
)


NBUF = 2


def _sc_body(loc_hbm, ct_tab_hbm, dur_tab_hbm, src_hbm, ct_hbm, dur_hbm,
             out_hbm, ct_v, durt_v, srcf_v, ctf_v, durf_v,
             gbuf0, gbuf1, sbuf0, sbuf1, g0, g1, s0, s1):
    wid = lax.axis_index("s") * NC + lax.axis_index("c")
    base_w = wid * ROWS_W
    gbufs, sbufs, gsems, ssems = [gbuf0, gbuf1], [sbuf0, sbuf1], [g0, g1], [s0, s1]
    pltpu.sync_copy(ct_tab_hbm, ct_v)
    pltpu.sync_copy(dur_tab_hbm, durt_v)
    pltpu.sync_copy(src_hbm.at[pl.ds(base_w, ROWS_W)], srcf_v)
    pltpu.sync_copy(ct_hbm.at[pl.ds(base_w, ROWS_W)], ctf_v)
    pltpu.sync_copy(dur_hbm.at[pl.ds(base_w, ROWS_W)], durf_v)

    def start_gather(c, b):
        pltpu.async_copy(loc_hbm.at[srcf_v.at[pl.ds(c * CHUNK, CHUNK)]],
                         gbufs[b], gsems[b])

    def wait_gather(c, b):
        pltpu.make_async_copy(loc_hbm.at[srcf_v.at[pl.ds(c * CHUNK, CHUNK)]],
                              gbufs[b], gsems[b]).wait()

    for b in range(NBUF):
        start_gather(b, b)

    def outer(i, carry):
        c0 = i * NBUF
        for b in range(NBUF):
            c = c0 + b
            wait_gather(c, b)

            @pl.when(c >= NBUF)
            def _():
                pltpu.make_async_copy(
                    sbufs[b], out_hbm.at[pl.ds(base_w, CHUNK)], ssems[b]).wait()

            def group_body(g, gcarry):
                gb = c * CHUNK + g * 16
                src16 = srcf_v[pl.ds(gb, 16)]
                ct16 = ctf_v[pl.ds(gb, 16)]
                dur16 = durf_v[pl.ds(gb, 16)]
                pad = src16 == 0
                rows16 = g * 16 + lax.iota(jnp.int32, 16)
                for d in range(D):
                    dsp = jnp.full((16,), d, jnp.int32)
                    s = (plsc.load_gather(ct_v, [ct16, dsp])
                         + plsc.load_gather(durt_v, [dur16, dsp]))
                    loc = plsc.load_gather(gbufs[b], [rows16, dsp])
                    val = s + jnp.where(pad, 0.0, loc)
                    plsc.store_scatter(sbufs[b], [rows16, dsp], val)
                return gcarry

            lax.fori_loop(0, GROUPS, group_body, 0)
            pltpu.async_copy(
                sbufs[b], out_hbm.at[pl.ds(base_w + c * CHUNK, CHUNK)], ssems[b])

            @pl.when(c + NBUF < NCHUNK)
            def _():
                start_gather(c + NBUF, b)
        return carry

    lax.fori_loop(0, NCHUNK // NBUF, outer, 0)
    for b in range(NBUF):
        pltpu.make_async_copy(
            sbufs[b], out_hbm.at[pl.ds(base_w, CHUNK)], ssems[b]).wait()


_sc_embed = functools.partial(
    pl.kernel,
    out_type=jax.ShapeDtypeStruct((N, D), jnp.float32),
    mesh=plsc.VectorSubcoreMesh(core_axis_name="c", subcore_axis_name="s"),
    compiler_params=pltpu.CompilerParams(needs_layout_passes=False,
                                         use_tc_tiling_on_sc=False),
    scratch_types=[
        pltpu.VMEM((CT_ROWS, D), jnp.float32),
        pltpu.VMEM((96, D), jnp.float32),
        pltpu.VMEM((ROWS_W,), jnp.int32),
        pltpu.VMEM((ROWS_W,), jnp.int32),
        pltpu.VMEM((ROWS_W,), jnp.int32),
        pltpu.VMEM((CHUNK, D), jnp.float32),
        pltpu.VMEM((CHUNK, D), jnp.float32),
        pltpu.VMEM((CHUNK, D), jnp.float32),
        pltpu.VMEM((CHUNK, D), jnp.float32),
        pltpu.SemaphoreType.DMA,
        pltpu.SemaphoreType.DMA,
        pltpu.SemaphoreType.DMA,
        pltpu.SemaphoreType.DMA,
    ],
)(_sc_body)


def kernel(src, time, weekday, duration, emb_loc, minute_embed, hour_embed,
           weekday_embed, emb_duration):
    ct_idx, ct_tab4 = _prep(time.astype(jnp.int32), weekday.astype(jnp.int32),
                            hour_embed, minute_embed, weekday_embed)
    ct_tab = ct_tab4.reshape(CT_ROWS, D)
    out = _sc_embed(emb_loc, ct_tab, emb_duration,
                    src.reshape(N).astype(jnp.int32),
                    ct_idx.reshape(N),
                    duration.reshape(N).astype(jnp.int32))
    return out.reshape(SEQ, B, D)


# speedup vs baseline: 2.1920x; 1.1976x over previous
import functools

import jax
import jax.numpy as jnp
from jax import lax
from jax.experimental import pallas as pl
from jax.experimental.pallas import tpu as pltpu
from jax.experimental.pallas import tpu_sc as plsc

SEQ, B, D = 200, 1024, 64
N = SEQ * B
NC, NS = 2, 16
NW = NC * NS
ROWS_W = N // NW
CHUNK = 128
NCHUNK = ROWS_W // CHUNK
GROUPS = CHUNK // 16
CT_ROWS = 24 * 4 * 7


def _prep_body(time_ref, wd_ref, hour_ref, minute_ref, wde_ref,
               ct_idx_ref, ct_tab_ref):
    ct_idx_ref[...] = time_ref[...] * 7 + wd_ref[...]
    h = hour_ref[...]
    mi = minute_ref[...]
    w = wde_ref[...]
    ct_tab_ref[...] = (h[:, None, None, :] + mi[None, :, None, :]
                       + w[None, None, :, :])


_prep = pl.pallas_call(
    _prep_body,
    out_shape=(
        jax.ShapeDtypeStruct((SEQ, B), jnp.int32),
        jax.ShapeDtypeStruct((24, 4, 7, D), jnp.float32),
    ),
)


NBUF = 2


def _sc_body(loc_hbm, ct_tab_hbm, dur_tab_hbm, src_hbm, ct_hbm, dur_hbm,
             out_hbm, ct_v, durt_v, srcf_v, ctf_v, durf_v,
             gbuf0, gbuf1, sbuf0, sbuf1, g0, g1, s0, s1):
    wid = lax.axis_index("s") * NC + lax.axis_index("c")
    base_w = wid * ROWS_W
    gbufs, sbufs, gsems, ssems = [gbuf0, gbuf1], [sbuf0, sbuf1], [g0, g1], [s0, s1]
    pltpu.sync_copy(ct_tab_hbm, ct_v)
    pltpu.sync_copy(dur_tab_hbm, durt_v)
    pltpu.sync_copy(src_hbm.at[pl.ds(base_w, ROWS_W)], srcf_v)
    pltpu.sync_copy(ct_hbm.at[pl.ds(base_w, ROWS_W)], ctf_v)
    pltpu.sync_copy(dur_hbm.at[pl.ds(base_w, ROWS_W)], durf_v)

    def start_gather(c, b):
        pltpu.async_copy(loc_hbm.at[srcf_v.at[pl.ds(c * CHUNK, CHUNK)]],
                         gbufs[b], gsems[b])

    def wait_gather(c, b):
        pltpu.make_async_copy(loc_hbm.at[srcf_v.at[pl.ds(c * CHUNK, CHUNK)]],
                              gbufs[b], gsems[b]).wait()

    for b in range(NBUF):
        start_gather(b, b)

    def outer(i, carry):
        c0 = i * NBUF
        for b in range(NBUF):
            c = c0 + b
            wait_gather(c, b)

            @pl.when(c >= NBUF)
            def _():
                pltpu.make_async_copy(
                    sbufs[b], out_hbm.at[pl.ds(base_w, CHUNK)], ssems[b]).wait()

            @plsc.parallel_loop(0, GROUPS)
            def group_body(g):
                gb = c * CHUNK + g * 16
                src16 = srcf_v[pl.ds(gb, 16)]
                ct16 = ctf_v[pl.ds(gb, 16)]
                dur16 = durf_v[pl.ds(gb, 16)]
                pad = src16 == 0
                rows16 = g * 16 + lax.iota(jnp.int32, 16)
                for d in range(D):
                    dsp = jnp.full((16,), d, jnp.int32)
                    s = (plsc.load_gather(ct_v, [ct16, dsp])
                         + plsc.load_gather(durt_v, [dur16, dsp]))
                    loc = plsc.load_gather(gbufs[b], [rows16, dsp])
                    val = s + jnp.where(pad, 0.0, loc)
                    plsc.store_scatter(sbufs[b], [rows16, dsp], val)
            pltpu.async_copy(
                sbufs[b], out_hbm.at[pl.ds(base_w + c * CHUNK, CHUNK)], ssems[b])

            @pl.when(c + NBUF < NCHUNK)
            def _():
                start_gather(c + NBUF, b)
        return carry

    lax.fori_loop(0, NCHUNK // NBUF, outer, 0)
    for b in range(NBUF):
        pltpu.make_async_copy(
            sbufs[b], out_hbm.at[pl.ds(base_w, CHUNK)], ssems[b]).wait()


_sc_embed = functools.partial(
    pl.kernel,
    out_type=jax.ShapeDtypeStruct((N, D), jnp.float32),
    mesh=plsc.VectorSubcoreMesh(core_axis_name="c", subcore_axis_name="s"),
    compiler_params=pltpu.CompilerParams(needs_layout_passes=False,
                                         use_tc_tiling_on_sc=False),
    scratch_types=[
        pltpu.VMEM((CT_ROWS, D), jnp.float32),
        pltpu.VMEM((96, D), jnp.float32),
        pltpu.VMEM((ROWS_W,), jnp.int32),
        pltpu.VMEM((ROWS_W,), jnp.int32),
        pltpu.VMEM((ROWS_W,), jnp.int32),
        pltpu.VMEM((CHUNK, D), jnp.float32),
        pltpu.VMEM((CHUNK, D), jnp.float32),
        pltpu.VMEM((CHUNK, D), jnp.float32),
        pltpu.VMEM((CHUNK, D), jnp.float32),
        pltpu.SemaphoreType.DMA,
        pltpu.SemaphoreType.DMA,
        pltpu.SemaphoreType.DMA,
        pltpu.SemaphoreType.DMA,
    ],
)(_sc_body)


def kernel(src, time, weekday, duration, emb_loc, minute_embed, hour_embed,
           weekday_embed, emb_duration):
    ct_idx, ct_tab4 = _prep(time.astype(jnp.int32), weekday.astype(jnp.int32),
                            hour_embed, minute_embed, weekday_embed)
    ct_tab = ct_tab4.reshape(CT_ROWS, D)
    out = _sc_embed(emb_loc, ct_tab, emb_duration,
                    src.reshape(N).astype(jnp.int32),
                    ct_idx.reshape(N),
                    duration.reshape(N).astype(jnp.int32))
    return out.reshape(SEQ, B, D)


# speedup vs baseline: 3.6298x; 1.6560x over previous
import functools

import jax
import jax.numpy as jnp
from jax import lax
from jax.experimental import pallas as pl
from jax.experimental.pallas import tpu as pltpu
from jax.experimental.pallas import tpu_sc as plsc

SEQ, B, D = 200, 1024, 64
N = SEQ * B
NC, NS = 2, 16
NW = NC * NS
ROWS_W = N // NW
CHUNK = 128
NCHUNK = ROWS_W // CHUNK
GROUPS = CHUNK // 16
CT_ROWS = 24 * 4 * 7


def _prep_body(time_ref, wd_ref, hour_ref, minute_ref, wde_ref,
               ct_idx_ref, ct_tab_ref):
    ct_idx_ref[...] = time_ref[...] * 7 + wd_ref[...]
    h = hour_ref[...]
    mi = minute_ref[...]
    w = wde_ref[...]
    ct_tab_ref[...] = (h[:, None, None, :] + mi[None, :, None, :]
                       + w[None, None, :, :])


_prep = pl.pallas_call(
    _prep_body,
    out_shape=(
        jax.ShapeDtypeStruct((SEQ, B), jnp.int32),
        jax.ShapeDtypeStruct((24, 4, 7, D), jnp.float32),
    ),
)


NBUF = 2


def _sc_body(loc_hbm, ct_tab_hbm, dur_tab_hbm, src_hbm, ct_hbm, dur_hbm,
             out_hbm, ct_v, durt_v, srcf_v, ctf_v, durf_v,
             gbuf0, gbuf1, sbuf0, sbuf1, g0, g1, s0, s1):
    wid = lax.axis_index("s") * NC + lax.axis_index("c")
    base_w = wid * ROWS_W
    gbufs, sbufs, gsems, ssems = [gbuf0, gbuf1], [sbuf0, sbuf1], [g0, g1], [s0, s1]
    pltpu.sync_copy(ct_tab_hbm, ct_v)
    pltpu.sync_copy(dur_tab_hbm, durt_v)
    pltpu.sync_copy(src_hbm.at[pl.ds(base_w, ROWS_W)], srcf_v)
    pltpu.sync_copy(ct_hbm.at[pl.ds(base_w, ROWS_W)], ctf_v)
    pltpu.sync_copy(dur_hbm.at[pl.ds(base_w, ROWS_W)], durf_v)

    def start_gather(c, b):
        pltpu.async_copy(loc_hbm.at[srcf_v.at[pl.ds(c * CHUNK, CHUNK)]],
                         gbufs[b], gsems[b])

    def wait_gather(c, b):
        pltpu.make_async_copy(loc_hbm.at[srcf_v.at[pl.ds(c * CHUNK, CHUNK)]],
                              gbufs[b], gsems[b]).wait()

    for b in range(NBUF):
        start_gather(b, b)

    def outer(i, carry):
        c0 = i * NBUF
        for b in range(NBUF):
            c = c0 + b
            wait_gather(c, b)

            @pl.when(c >= NBUF)
            def _():
                pltpu.make_async_copy(
                    sbufs[b], out_hbm.at[pl.ds(base_w, CHUNK)], ssems[b]).wait()

            @plsc.parallel_loop(0, GROUPS, unroll=2)
            def group_body(g):
                gb = c * CHUNK + g * 16
                ct16 = ctf_v[pl.ds(gb, 16)]
                dur16 = durf_v[pl.ds(gb, 16)]
                src16 = srcf_v[pl.ds(gb, 16)]
                keep16 = jnp.where(src16 == 0, 0.0, 1.0)
                for j in range(16):
                    r = g * 16 + j
                    ct_r, dur_r, keep = ct16[j], dur16[j], keep16[j]
                    for k in range(D // 16):
                        a = ct_v[ct_r, pl.ds(k * 16, 16)]
                        t = durt_v[dur_r, pl.ds(k * 16, 16)]
                        l = gbufs[b][r, pl.ds(k * 16, 16)]
                        sbufs[b][r, pl.ds(k * 16, 16)] = a + t + l * keep
            pltpu.async_copy(
                sbufs[b], out_hbm.at[pl.ds(base_w + c * CHUNK, CHUNK)], ssems[b])

            @pl.when(c + NBUF < NCHUNK)
            def _():
                start_gather(c + NBUF, b)
        return carry

    lax.fori_loop(0, NCHUNK // NBUF, outer, 0)
    for b in range(NBUF):
        pltpu.make_async_copy(
            sbufs[b], out_hbm.at[pl.ds(base_w, CHUNK)], ssems[b]).wait()


_sc_embed = functools.partial(
    pl.kernel,
    out_type=jax.ShapeDtypeStruct((N, D), jnp.float32),
    mesh=plsc.VectorSubcoreMesh(core_axis_name="c", subcore_axis_name="s"),
    compiler_params=pltpu.CompilerParams(needs_layout_passes=False,
                                         use_tc_tiling_on_sc=False),
    scratch_types=[
        pltpu.VMEM((CT_ROWS, D), jnp.float32),
        pltpu.VMEM((96, D), jnp.float32),
        pltpu.VMEM((ROWS_W,), jnp.int32),
        pltpu.VMEM((ROWS_W,), jnp.int32),
        pltpu.VMEM((ROWS_W,), jnp.int32),
        pltpu.VMEM((CHUNK, D), jnp.float32),
        pltpu.VMEM((CHUNK, D), jnp.float32),
        pltpu.VMEM((CHUNK, D), jnp.float32),
        pltpu.VMEM((CHUNK, D), jnp.float32),
        pltpu.SemaphoreType.DMA,
        pltpu.SemaphoreType.DMA,
        pltpu.SemaphoreType.DMA,
        pltpu.SemaphoreType.DMA,
    ],
)(_sc_body)


def kernel(src, time, weekday, duration, emb_loc, minute_embed, hour_embed,
           weekday_embed, emb_duration):
    ct_idx, ct_tab4 = _prep(time.astype(jnp.int32), weekday.astype(jnp.int32),
                            hour_embed, minute_embed, weekday_embed)
    ct_tab = ct_tab4.reshape(CT_ROWS, D)
    out = _sc_embed(emb_loc, ct_tab, emb_duration,
                    src.reshape(N).astype(jnp.int32),
                    ct_idx.reshape(N),
                    duration.reshape(N).astype(jnp.int32))
    return out.reshape(SEQ, B, D)
